# baseline (speedup 1.0000x reference)
import functools

import jax
import jax.numpy as jnp
from jax import lax
from jax.experimental import pallas as pl
from jax.experimental.pallas import tpu as pltpu
from jax.experimental.pallas import tpu_sc as plsc

VOC_SIZE = 1000000
EMBED_DIM = 32

NUM_WORKERS = 32
T_STEPS = 200
BL = 128
NBUF = 4
TBUF = 2


def _make_gather():
    mesh = plsc.VectorSubcoreMesh(core_axis_name="c", subcore_axis_name="s")

    @functools.partial(
        pl.kernel,
        mesh=mesh,
        compiler_params=pltpu.CompilerParams(use_tc_tiling_on_sc=False,
                                             needs_layout_passes=False),
        out_type=jax.ShapeDtypeStruct((T_STEPS, 4, NUM_WORKERS, 8, BL),
                                      jnp.float32),
        scratch_types=[
            pltpu.VMEM((T_STEPS, BL), jnp.int32),
            [pltpu.VMEM((BL, EMBED_DIM), jnp.float32)] * NBUF,
            [pltpu.VMEM((4 * 8, BL + 1), jnp.float32)] * TBUF,
            [pltpu.SemaphoreType.DMA] * NBUF,
            [pltpu.SemaphoreType.DMA] * TBUF,
        ],
    )
    def gather_kernel(table_hbm, idx_hbm, out_hbm, idx_v, rows_v, tile_v,
                      gsems, tsems):
        wid = lax.axis_index("s") * 2 + lax.axis_index("c")
        pltpu.sync_copy(idx_hbm.at[wid], idx_v)

        lane = lax.iota(jnp.int32, 16)
        lane_h = [lane, lane + 16]

        def gather_desc(t, br):
            return pltpu.make_async_copy(
                table_hbm.at[idx_v.at[t]], rows_v[br], gsems[br]
            )

        def store_desc(t, tb, c_hi):
            return pltpu.make_async_copy(
                tile_v[tb].at[pl.ds(8 * c_hi, 8), pl.ds(0, BL)],
                out_hbm.at[t, c_hi, wid], tsems[tb]
            )

        for t0 in range(NBUF):
            gather_desc(t0, t0).start()

        def body(g, carry):
            for br in range(NBUF):
                t = g * NBUF + br
                tb = br % TBUF
                gather_desc(t, br).wait()

                @pl.when(t >= TBUF)
                def _():
                    for c_hi in range(4):
                        store_desc(t - TBUF, tb, c_hi).wait()

                for l0 in range(0, BL, 16):
                    vs = []
                    for l in range(l0, l0 + 16):
                        for h in range(2):
                            vs.append(rows_v[br][l, pl.ds(16 * h, 16)])
                    for i, l in enumerate(range(l0, l0 + 16)):
                        l_vec = jnp.full((16,), l, jnp.int32)
                        for h in range(2):
                            plsc.store_scatter(
                                tile_v[tb], [lane_h[h], l_vec],
                                vs[2 * i + h]
                            )

                for c_hi in range(4):
                    store_desc(t, tb, c_hi).start()

                @pl.when(t + NBUF < T_STEPS)
                def _():
                    gather_desc(t + NBUF, br).start()

            return carry

        lax.fori_loop(0, T_STEPS // NBUF, body, 0)

        for dt in range(TBUF):
            t = T_STEPS - TBUF + dt
            for c_hi in range(4):
                store_desc(t, t % TBUF, c_hi).wait()

    return gather_kernel


_gather = _make_gather()


def kernel(X, table):
    idx3 = X.reshape(NUM_WORKERS, BL, T_STEPS).transpose(0, 2, 1)
    idx3 = idx3.astype(jnp.int32)
    out6 = _gather(table, idx3)
    out = out6.transpose(2, 4, 0, 1, 3).reshape(4096, T_STEPS, EMBED_DIM)
    return out


# speedup vs baseline: 1.0313x; 1.0313x over previous
import functools

import jax
import jax.numpy as jnp
from jax import lax
from jax.experimental import pallas as pl
from jax.experimental.pallas import tpu as pltpu
from jax.experimental.pallas import tpu_sc as plsc

VOC_SIZE = 1000000
EMBED_DIM = 32

NUM_WORKERS = 32
T_STEPS = 200
BL = 128
NBUF = 4
TBUF = 2


def _make_gather():
    mesh = plsc.VectorSubcoreMesh(core_axis_name="c", subcore_axis_name="s")

    @functools.partial(
        pl.kernel,
        mesh=mesh,
        compiler_params=pltpu.CompilerParams(use_tc_tiling_on_sc=False,
                                             needs_layout_passes=False),
        out_type=jax.ShapeDtypeStruct((T_STEPS, 4, NUM_WORKERS, 8, BL),
                                      jnp.float32),
        scratch_types=[
            pltpu.VMEM((T_STEPS, BL), jnp.int32),
            [pltpu.VMEM((BL, EMBED_DIM), jnp.float32)] * NBUF,
            [pltpu.VMEM((4 * 8, BL + 1), jnp.float32)] * TBUF,
            [pltpu.SemaphoreType.DMA] * NBUF,
            [pltpu.SemaphoreType.DMA] * TBUF,
        ],
    )
    def gather_kernel(table_hbm, idx_hbm, out_hbm, idx_v, rows_v, tile_v,
                      gsems, tsems):
        wid = lax.axis_index("s") * 2 + lax.axis_index("c")
        pltpu.sync_copy(idx_hbm.at[wid], idx_v)

        lane = lax.iota(jnp.int32, 16)
        lane_h = [lane, lane + 16]

        def gather_desc(t, br):
            return pltpu.make_async_copy(
                table_hbm.at[idx_v.at[t]], rows_v[br], gsems[br]
            )

        def store_desc(t, tb, c_hi):
            return pltpu.make_async_copy(
                tile_v[tb].at[pl.ds(8 * c_hi, 8), pl.ds(0, BL)],
                out_hbm.at[t, c_hi, wid], tsems[tb]
            )

        for t0 in range(NBUF):
            gather_desc(t0, t0).start()

        def body(g, carry):
            for br in range(NBUF):
                t = g * NBUF + br
                tb = br % TBUF
                gather_desc(t, br).wait()

                @pl.when(t >= TBUF)
                def _():
                    for c_hi in range(4):
                        store_desc(t - TBUF, tb, c_hi).wait()

                for l0 in range(0, BL, 8):
                    vs = []
                    for l in range(l0, l0 + 8):
                        for h in range(2):
                            vs.append(rows_v[br][l, pl.ds(16 * h, 16)])
                    for i, l in enumerate(range(l0, l0 + 8)):
                        l_vec = jnp.full((16,), l, jnp.int32)
                        for h in range(2):
                            plsc.store_scatter(
                                tile_v[tb], [lane_h[h], l_vec],
                                vs[2 * i + h]
                            )

                for c_hi in range(4):
                    store_desc(t, tb, c_hi).start()

                @pl.when(t + NBUF < T_STEPS)
                def _():
                    gather_desc(t + NBUF, br).start()

            return carry

        lax.fori_loop(0, T_STEPS // NBUF, body, 0)

        for dt in range(TBUF):
            t = T_STEPS - TBUF + dt
            for c_hi in range(4):
                store_desc(t, t % TBUF, c_hi).wait()

    return gather_kernel


_gather = _make_gather()


def kernel(X, table):
    idx3 = X.reshape(NUM_WORKERS, BL, T_STEPS).transpose(0, 2, 1)
    idx3 = idx3.astype(jnp.int32)
    out6 = _gather(table, idx3)
    out = out6.transpose(2, 4, 0, 1, 3).reshape(4096, T_STEPS, EMBED_DIM)
    return out
